# baseline (speedup 1.0000x reference)
import functools

import jax
import jax.numpy as jnp
import numpy as np
from jax import lax
from jax.experimental import pallas as pl
from jax.experimental.pallas import tpu as pltpu
from jax.experimental.pallas import tpu_sc as plsc

N = 10000
NP = 10240
H = 2
E0 = 160000
ESL = E0 + N
EP = 172032
EPD = 163840
FW = 64
EB = 64
NB = EP // (16 * EB)
NBD = EPD // (32 * EB)
RPT = NP // 16
NEG = -1e9
_PK = plsc.PackFormat.INTERLEAVED

_MESH = dict(core_axis_name="c", subcore_axis_name="s")
_SC_PARAMS = pltpu.CompilerParams(
    use_tc_tiling_on_sc=False, needs_layout_passes=False)


def _tc_body(x_ref, w_ref, as_ref, ad_ref, tbl_ref, asrc_ref, adst_ref,
             *, nk, relu):
    xb = x_ref[...]
    if relu:
        xb = jnp.maximum(xb, 0.0)
    xw = jnp.dot(xb, w_ref[...], preferred_element_type=jnp.float32)
    for k in range(nk):
        tbl_ref[k, :, :] = xw[:, k * FW:(k + 1) * FW].astype(jnp.bfloat16)
    asrc_ref[...] = jnp.dot(xw, as_ref[...], preferred_element_type=jnp.float32)
    adst_ref[...] = jnp.dot(xw, ad_ref[...], preferred_element_type=jnp.float32)


def _tc_layer(x_p, w_mat, att_s, att_d, *, nk, relu):
    inw = x_p.shape[1]
    f = w_mat.shape[1]
    grid = NP // 256
    return pl.pallas_call(
        functools.partial(_tc_body, nk=nk, relu=relu),
        grid=(grid,),
        in_specs=[
            pl.BlockSpec((256, inw), lambda i: (i, 0)),
            pl.BlockSpec((inw, f), lambda i: (0, 0)),
            pl.BlockSpec((f, H), lambda i: (0, 0)),
            pl.BlockSpec((f, H), lambda i: (0, 0)),
        ],
        out_specs=[
            pl.BlockSpec((nk, 256, FW), lambda i: (0, i, 0)),
            pl.BlockSpec((256, H), lambda i: (i, 0)),
            pl.BlockSpec((256, H), lambda i: (i, 0)),
        ],
        out_shape=[
            jax.ShapeDtypeStruct((nk, NP, FW), jnp.bfloat16),
            jax.ShapeDtypeStruct((NP, H), jnp.float32),
            jax.ShapeDtypeStruct((NP, H), jnp.float32),
        ],
    )(x_p, w_mat, att_s, att_d)


def _zero2d(ref, nrows, nvec):
    z = jnp.zeros((16,), jnp.float32)

    def body(r, c):
        for j in range(nvec):
            ref[r, pl.ds(j * 16, 16)] = z
        return c

    lax.fori_loop(0, nrows, body, 0)


def _sc_layer_body(src2d, dst2d, asrc_t, adst_t, tbl, bias2d, out_hbm,
                   idx_s, idx_d, atbl_s, atbl_d, wbuf, bf0, bf1, f0, f1,
                   w0, w1, orows, bias_v,
                   gsem0, gsem1, ssem0, ssem1, dsem0, dsem1, acc, dacc,
                   *, cpc, out_bf):
    cid = lax.axis_index("c")
    sid = lax.axis_index("s")

    pltpu.sync_copy(src2d.at[sid], idx_s)
    pltpu.sync_copy(dst2d.at[sid], idx_d)
    pltpu.sync_copy(asrc_t.at[pl.ds(cid * NP, NP)], atbl_s)
    pltpu.sync_copy(adst_t.at[pl.ds(cid * NP, NP)], atbl_d)

    _zero2d(f0, EB, FW // 16)
    _zero2d(w0, EB, 1)
    for kb in range(RPT // EB):
        pltpu.sync_copy(f0, acc.at[pl.ds(sid * RPT + kb * EB, EB)])
        pltpu.sync_copy(w0, dacc.at[pl.ds(sid * RPT + kb * EB, EB)])

    def wb(b, c):
        for i in range(EB // 16):
            sv = idx_s[b, pl.ds(i * 16, 16)]
            dv = idx_d[b, pl.ds(i * 16, 16)]
            al = (plsc.load_gather(atbl_s, [sv])
                  + plsc.load_gather(atbl_d, [dv]))
            al = jnp.maximum(al, 0.2 * al)
            wbuf[b, pl.ds(i * 16, 16)] = jnp.exp(al)
        return c

    lax.fori_loop(0, NB, wb, 0)
    plsc.subcore_barrier()

    npair = NB // 2

    for cc in range(cpc):
        wd = cc == 0
        k_dyn = cid * cpc + cc
        tblk = tbl.at[k_dyn]
        pltpu.sync_copy(bias2d.at[pl.ds(k_dyn * FW, FW)], bias_v)

        def scale(b, bbuf, fbuf, wr):
            def inner(i, c):
                wv = wbuf[b, pl.ds(i * 16, 16)]
                for l in range(16):
                    e = i * 16 + l
                    w = wv[l]
                    for j in range(FW // 32):
                        ab = bbuf[e, pl.ds(j * 32, 32)]
                        va, vb = plsc.unpack(ab, format=_PK)
                        fbuf[e, pl.ds(j * 32, 16)] = va * w
                        fbuf[e, pl.ds(j * 32 + 16, 16)] = vb * w
                    if wd:
                        wr[e, :] = jnp.zeros((16,), jnp.float32) + w
                return c

            lax.fori_loop(0, EB // 16, inner, 0)

        def gst(b, buf, sem):
            pltpu.async_copy(tblk.at[idx_s.at[b]], buf, sem)

        def gwt(b, buf, sem):
            pltpu.make_async_copy(tblk.at[idx_s.at[b]], buf, sem).wait()

        def sst(b, buf, wr, sems):
            pltpu.async_copy(buf, acc.at[idx_d.at[b]], sems[0], add=True)
            if wd:
                pltpu.async_copy(wr, dacc.at[idx_d.at[b]], sems[1], add=True)

        def swt(b, buf, wr, sems):
            pltpu.make_async_copy(buf, acc.at[idx_d.at[b]], sems[0]).wait()
            if wd:
                pltpu.make_async_copy(wr, dacc.at[idx_d.at[b]], sems[1]).wait()

        gst(0, bf0, gsem0)

        def pair(g, c):
            b0 = 2 * g
            b1 = b0 + 1

            @pl.when(g > 0)
            def _():
                swt(b1, f1, w1, (ssem1, dsem1))

            gst(b1, bf1, gsem1)
            gwt(b0, bf0, gsem0)
            scale(b0, bf0, f0, w0)
            sst(b0, f0, w0, (ssem0, dsem0))
            gwt(b1, bf1, gsem1)
            scale(b1, bf1, f1, w1)
            sst(b1, f1, w1, (ssem1, dsem1))

            @pl.when(g < npair - 1)
            def _():
                swt(b0, f0, w0, (ssem0, dsem0))
                gst(b0 + 2, bf0, gsem0)

            return c

        lax.fori_loop(0, npair, pair, 0)
        swt(0, f0, w0, (ssem0, dsem0))
        swt(0, f1, w1, (ssem1, dsem1))
        plsc.subcore_barrier()

        for kb in range(RPT // EB):
            r0 = sid * RPT + kb * EB
            pltpu.sync_copy(acc.at[pl.ds(r0, EB)], f0)
            pltpu.sync_copy(dacc.at[pl.ds(r0, EB)], w0)

            def div(r, c):
                dv = w0[r, :]
                rcpv = 1.0 / (dv + 1e-16)
                rcp = rcpv[0]
                if out_bf:
                    for j in range(FW // 32):
                        va = (f0[r, pl.ds(j * 32, 16)] * rcp
                              + bias_v[pl.ds(j * 32, 16)])
                        vb = (f0[r, pl.ds(j * 32 + 16, 16)] * rcp
                              + bias_v[pl.ds(j * 32 + 16, 16)])
                        orows[r, pl.ds(j * 32, 32)] = plsc.pack(
                            va, vb, format=_PK)
                else:
                    for j in range(FW // 16):
                        orows[r, pl.ds(j * 16, 16)] = (
                            f0[r, pl.ds(j * 16, 16)] * rcp
                            + bias_v[pl.ds(j * 16, 16)])
                return c

            lax.fori_loop(0, EB, div, 0)
            pltpu.sync_copy(
                orows, out_hbm.at[pl.ds(r0, EB), pl.ds(k_dyn * FW, FW)])

        if cc + 1 < cpc:
            _zero2d(f0, EB, FW // 16)
            for kb in range(RPT // EB):
                pltpu.sync_copy(f0, acc.at[pl.ds(sid * RPT + kb * EB, EB)])
            plsc.subcore_barrier()


def _sc_layer(src2d, dst2d, asrc_t, adst_t, tbl, bias2d, *, nk, out_bf):
    cpc = nk // 2
    odt = jnp.bfloat16 if out_bf else jnp.float32
    fn = functools.partial(
        pl.kernel,
        functools.partial(_sc_layer_body, cpc=cpc, out_bf=out_bf),
        out_type=jax.ShapeDtypeStruct((NP, nk * FW), odt),
        mesh=plsc.VectorSubcoreMesh(**_MESH),
        scratch_types=[
            pltpu.VMEM((NB, EB), jnp.int32),
            pltpu.VMEM((NB, EB), jnp.int32),
            pltpu.VMEM((NP,), jnp.float32),
            pltpu.VMEM((NP,), jnp.float32),
            pltpu.VMEM((NB, EB), jnp.float32),
            pltpu.VMEM((EB, FW), jnp.bfloat16),
            pltpu.VMEM((EB, FW), jnp.bfloat16),
            pltpu.VMEM((EB, FW), jnp.float32),
            pltpu.VMEM((EB, FW), jnp.float32),
            pltpu.VMEM((EB, 16), jnp.float32),
            pltpu.VMEM((EB, 16), jnp.float32),
            pltpu.VMEM((EB, FW), odt),
            pltpu.VMEM((FW,), jnp.float32),
            pltpu.SemaphoreType.DMA,
            pltpu.SemaphoreType.DMA,
            pltpu.SemaphoreType.DMA,
            pltpu.SemaphoreType.DMA,
            pltpu.SemaphoreType.DMA,
            pltpu.SemaphoreType.DMA,
            pltpu.VMEM_SHARED((NP, FW), jnp.float32),
            pltpu.VMEM_SHARED((NP, 16), jnp.float32),
        ],
        compiler_params=_SC_PARAMS,
    )()
    return fn(src2d, dst2d, asrc_t, adst_t, tbl, bias2d)


def _sc_decode_body(z_hbm, s2d, d2d, out_hbm, sidx, didx, sr0, dr0, sr1, dr1,
                    pbuf, obuf, gsem0, gsem1):
    cid = lax.axis_index("c")
    sid = lax.axis_index("s")
    g = sid * 2 + cid
    pltpu.sync_copy(s2d.at[g], sidx)
    pltpu.sync_copy(d2d.at[g], didx)
    lanes = lax.iota(jnp.int32, 16)

    def gst(b, sbuf, dbuf, sem):
        pltpu.async_copy(z_hbm.at[sidx.at[b]], sbuf, sem)
        pltpu.async_copy(z_hbm.at[didx.at[b]], dbuf, sem)

    def gwt(b, sbuf, dbuf, sem):
        pltpu.make_async_copy(z_hbm.at[sidx.at[b]], sbuf, sem).wait()
        pltpu.make_async_copy(z_hbm.at[didx.at[b]], dbuf, sem).wait()

    def dot(b, sbuf, dbuf):
        def grp(i, c):
            for l in range(16):
                e = i * 16 + l
                acc = None
                for j in range(256 // 32):
                    sa, sb = plsc.unpack(sbuf[e, pl.ds(j * 32, 32)],
                                         format=_PK)
                    da, db = plsc.unpack(dbuf[e, pl.ds(j * 32, 32)],
                                         format=_PK)
                    t = sa * da + sb * db
                    acc = t if acc is None else acc + t
                pbuf[l, :] = acc
            tot = plsc.load_gather(pbuf, [lanes, jnp.zeros((16,), jnp.int32)])
            for j in range(1, 16):
                tot = tot + plsc.load_gather(
                    pbuf, [lanes, jnp.full((16,), j, jnp.int32)])
            obuf[pl.ds(b * EB + i * 16, 16)] = tot
            return c

        lax.fori_loop(0, EB // 16, grp, 0)

    npair = NBD // 2
    gst(0, sr0, dr0, gsem0)

    def pair(gp, c):
        b0 = 2 * gp
        b1 = b0 + 1
        gst(b1, sr1, dr1, gsem1)
        gwt(b0, sr0, dr0, gsem0)
        dot(b0, sr0, dr0)

        @pl.when(gp < npair - 1)
        def _():
            gst(b0 + 2, sr0, dr0, gsem0)

        gwt(b1, sr1, dr1, gsem1)
        dot(b1, sr1, dr1)
        return c

    lax.fori_loop(0, npair, pair, 0)
    pltpu.sync_copy(obuf, out_hbm.at[pl.ds(g * NBD * EB, NBD * EB)])


def _sc_decode(z, s2d, d2d):
    fn = functools.partial(
        pl.kernel,
        _sc_decode_body,
        out_type=jax.ShapeDtypeStruct((EPD,), jnp.float32),
        mesh=plsc.VectorSubcoreMesh(**_MESH),
        scratch_types=[
            pltpu.VMEM((NBD, EB), jnp.int32),
            pltpu.VMEM((NBD, EB), jnp.int32),
            pltpu.VMEM((EB, 256), jnp.bfloat16),
            pltpu.VMEM((EB, 256), jnp.bfloat16),
            pltpu.VMEM((EB, 256), jnp.bfloat16),
            pltpu.VMEM((EB, 256), jnp.bfloat16),
            pltpu.VMEM((16, 16), jnp.float32),
            pltpu.VMEM((NBD * EB,), jnp.float32),
            pltpu.SemaphoreType.DMA,
            pltpu.SemaphoreType.DMA,
        ],
        compiler_params=_SC_PARAMS,
    )()
    return fn(z, s2d, d2d)


def _block_diag_att(att):
    hh, c = att.shape
    m = jnp.zeros((hh * c, hh), jnp.float32)
    for h in range(hh):
        m = m.at[h * c:(h + 1) * c, h].set(att[h])
    return m


def _unpack_perm(n):
    blk = np.concatenate([np.arange(0, 32, 2), np.arange(1, 32, 2)])
    return (np.arange(n).reshape(-1, 32)[:, blk]).reshape(-1)


def kernel(x, edge_index, W1, att_src1, att_dst1, b1, W2, att_src2, att_dst2,
           b2):
    ei = edge_index.astype(jnp.int32)
    x_p = jnp.pad(x, ((0, NP - N), (0, 0)))
    loop = jnp.arange(N, dtype=jnp.int32)
    padc = jnp.full((EP - ESL,), N, jnp.int32)
    src2d = jnp.concatenate([ei[0], loop, padc]).reshape(16, NB, EB)
    dst2d = jnp.concatenate([ei[1], loop, padc]).reshape(16, NB, EB)
    p512 = _unpack_perm(512)
    p256 = _unpack_perm(256)

    tbl1, asrc1, adst1 = _tc_layer(
        x_p, W1, _block_diag_att(att_src1), _block_diag_att(att_dst1),
        nk=8, relu=False)
    asrc1t = asrc1.T.reshape(H * NP)
    adst1t = adst1.T.at[:, N:].set(NEG).reshape(H * NP)
    agg1 = _sc_layer(src2d, dst2d, asrc1t, adst1t, tbl1,
                     b1[p512], nk=8, out_bf=False)

    tbl2, asrc2, adst2 = _tc_layer(
        agg1, W2[p512, :], _block_diag_att(att_src2),
        _block_diag_att(att_dst2), nk=4, relu=True)
    asrc2t = asrc2.T.reshape(H * NP)
    adst2t = adst2.T.at[:, N:].set(NEG).reshape(H * NP)
    z = _sc_layer(src2d, dst2d, asrc2t, adst2t, tbl2,
                  b2[p256], nk=4, out_bf=True)

    padd = jnp.full((EPD - E0,), N, jnp.int32)
    s2d = jnp.concatenate([ei[0], padd]).reshape(32, NBD, EB)
    d2d = jnp.concatenate([ei[1], padd]).reshape(32, NBD, EB)
    scores = _sc_decode(z, s2d, d2d)
    return scores[:E0]


# speedup vs baseline: 1.2932x; 1.2932x over previous
import functools

import jax
import jax.numpy as jnp
import numpy as np
from jax import lax
from jax.experimental import pallas as pl
from jax.experimental.pallas import tpu as pltpu
from jax.experimental.pallas import tpu_sc as plsc

N = 10000
NP = 10240
H = 2
E0 = 160000
ESL = E0 + N
EP = 172032
EPD = 163840
FW = 64
EB = 128
NB = EP // (16 * EB)
NBD = EPD // (32 * EB)
RPT = NP // 16
NEG = -1e9
_PK = plsc.PackFormat.INTERLEAVED

_MESH = dict(core_axis_name="c", subcore_axis_name="s")
_SC_PARAMS = pltpu.CompilerParams(
    use_tc_tiling_on_sc=False, needs_layout_passes=False)


def _tc_body(x_ref, w_ref, as_ref, ad_ref, tbl_ref, asrc_ref, adst_ref,
             *, nk, relu):
    xb = x_ref[...]
    if relu:
        xb = jnp.maximum(xb, 0.0)
    xw = jnp.dot(xb, w_ref[...], preferred_element_type=jnp.float32)
    for k in range(nk):
        tbl_ref[k, :, :] = xw[:, k * FW:(k + 1) * FW]
    asrc_ref[...] = jnp.dot(xw, as_ref[...], preferred_element_type=jnp.float32)
    adst_ref[...] = jnp.dot(xw, ad_ref[...], preferred_element_type=jnp.float32)


def _tc_layer(x_p, w_mat, att_s, att_d, *, nk, relu):
    inw = x_p.shape[1]
    f = w_mat.shape[1]
    grid = NP // 256
    return pl.pallas_call(
        functools.partial(_tc_body, nk=nk, relu=relu),
        grid=(grid,),
        in_specs=[
            pl.BlockSpec((256, inw), lambda i: (i, 0)),
            pl.BlockSpec((inw, f), lambda i: (0, 0)),
            pl.BlockSpec((f, H), lambda i: (0, 0)),
            pl.BlockSpec((f, H), lambda i: (0, 0)),
        ],
        out_specs=[
            pl.BlockSpec((nk, 256, FW), lambda i: (0, i, 0)),
            pl.BlockSpec((256, H), lambda i: (i, 0)),
            pl.BlockSpec((256, H), lambda i: (i, 0)),
        ],
        out_shape=[
            jax.ShapeDtypeStruct((nk, NP, FW), jnp.float32),
            jax.ShapeDtypeStruct((NP, H), jnp.float32),
            jax.ShapeDtypeStruct((NP, H), jnp.float32),
        ],
    )(x_p, w_mat, att_s, att_d)


def _zero2d(ref, nrows, nvec):
    z = jnp.zeros((16,), jnp.float32)

    def body(r, c):
        for j in range(nvec):
            ref[r, pl.ds(j * 16, 16)] = z
        return c

    lax.fori_loop(0, nrows, body, 0)


def _sc_layer_body(src2d, dst2d, asrc_t, adst_t, tbl, bias2d, out_hbm,
                   idx_s, idx_d, atbl_s, atbl_d, wbuf, f0, f1,
                   w0, w1, orows, bias_v,
                   gsem0, gsem1, ssem0, ssem1, dsem0, dsem1, acc, dacc,
                   *, cpc, out_bf):
    cid = lax.axis_index("c")
    sid = lax.axis_index("s")

    pltpu.sync_copy(src2d.at[sid], idx_s)
    pltpu.sync_copy(dst2d.at[sid], idx_d)
    pltpu.sync_copy(asrc_t.at[pl.ds(cid * NP, NP)], atbl_s)
    pltpu.sync_copy(adst_t.at[pl.ds(cid * NP, NP)], atbl_d)

    _zero2d(f0, EB, FW // 16)
    _zero2d(w0, EB, 1)
    for kb in range(RPT // EB):
        pltpu.sync_copy(f0, acc.at[pl.ds(sid * RPT + kb * EB, EB)])
        pltpu.sync_copy(w0, dacc.at[pl.ds(sid * RPT + kb * EB, EB)])

    def wb(b, c):
        for i in range(EB // 16):
            sv = idx_s[b, pl.ds(i * 16, 16)]
            dv = idx_d[b, pl.ds(i * 16, 16)]
            al = (plsc.load_gather(atbl_s, [sv])
                  + plsc.load_gather(atbl_d, [dv]))
            al = jnp.maximum(al, 0.2 * al)
            wbuf[b, pl.ds(i * 16, 16)] = jnp.exp(al)
        return c

    lax.fori_loop(0, NB, wb, 0)
    plsc.subcore_barrier()

    npair = NB // 2

    for cc in range(cpc):
        wd = cc == 0
        k_dyn = cid * cpc + cc
        tblk = tbl.at[k_dyn]
        pltpu.sync_copy(bias2d.at[pl.ds(k_dyn * FW, FW)], bias_v)

        def scale(b, fbuf, wr):
            for i in range(EB // 16):
                wv = wbuf[b, pl.ds(i * 16, 16)]
                for l in range(16):
                    e = i * 16 + l
                    w = wv[l]
                    for j in range(FW // 16):
                        fbuf[e, pl.ds(j * 16, 16)] = (
                            fbuf[e, pl.ds(j * 16, 16)] * w)
                    if wd:
                        wr[e, :] = jnp.zeros((16,), jnp.float32) + w

        def gst(b, buf, sem):
            pltpu.async_copy(tblk.at[idx_s.at[b]], buf, sem)

        def gwt(b, buf, sem):
            pltpu.make_async_copy(tblk.at[idx_s.at[b]], buf, sem).wait()

        def sst(b, buf, wr, sems):
            pltpu.async_copy(buf, acc.at[idx_d.at[b]], sems[0], add=True)
            if wd:
                pltpu.async_copy(wr, dacc.at[idx_d.at[b]], sems[1], add=True)

        def swt(b, buf, wr, sems):
            pltpu.make_async_copy(buf, acc.at[idx_d.at[b]], sems[0]).wait()
            if wd:
                pltpu.make_async_copy(wr, dacc.at[idx_d.at[b]], sems[1]).wait()

        gst(0, f0, gsem0)

        def pair(g, c):
            b0 = 2 * g
            b1 = b0 + 1

            @pl.when(g > 0)
            def _():
                swt(b1, f1, w1, (ssem1, dsem1))

            gst(b1, f1, gsem1)
            gwt(b0, f0, gsem0)
            scale(b0, f0, w0)
            sst(b0, f0, w0, (ssem0, dsem0))
            gwt(b1, f1, gsem1)
            scale(b1, f1, w1)
            sst(b1, f1, w1, (ssem1, dsem1))

            @pl.when(g < npair - 1)
            def _():
                swt(b0, f0, w0, (ssem0, dsem0))
                gst(b0 + 2, f0, gsem0)

            return c

        lax.fori_loop(0, npair, pair, 0)
        swt(0, f0, w0, (ssem0, dsem0))
        swt(0, f1, w1, (ssem1, dsem1))
        plsc.subcore_barrier()

        for kb in range(RPT // EB):
            r0 = sid * RPT + kb * EB
            pltpu.sync_copy(acc.at[pl.ds(r0, EB)], f0)
            pltpu.sync_copy(dacc.at[pl.ds(r0, EB)], w0)

            def div(r, c):
                dv = w0[r, :]
                rcpv = 1.0 / (dv + 1e-16)
                rcp = rcpv[0]
                if out_bf:
                    for j in range(FW // 32):
                        va = (f0[r, pl.ds(j * 32, 16)] * rcp
                              + bias_v[pl.ds(j * 32, 16)])
                        vb = (f0[r, pl.ds(j * 32 + 16, 16)] * rcp
                              + bias_v[pl.ds(j * 32 + 16, 16)])
                        orows[r, pl.ds(j * 32, 32)] = plsc.pack(
                            va, vb, format=_PK)
                else:
                    for j in range(FW // 16):
                        f0[r, pl.ds(j * 16, 16)] = (
                            f0[r, pl.ds(j * 16, 16)] * rcp
                            + bias_v[pl.ds(j * 16, 16)])
                return c

            lax.fori_loop(0, EB, div, 0)
            osrc = orows if out_bf else f0
            pltpu.sync_copy(
                osrc, out_hbm.at[pl.ds(r0, EB), pl.ds(k_dyn * FW, FW)])

        if cc + 1 < cpc:
            _zero2d(f0, EB, FW // 16)
            for kb in range(RPT // EB):
                pltpu.sync_copy(f0, acc.at[pl.ds(sid * RPT + kb * EB, EB)])
            plsc.subcore_barrier()


def _sc_layer(src2d, dst2d, asrc_t, adst_t, tbl, bias2d, *, nk, out_bf):
    cpc = nk // 2
    odt = jnp.bfloat16 if out_bf else jnp.float32
    fn = functools.partial(
        pl.kernel,
        functools.partial(_sc_layer_body, cpc=cpc, out_bf=out_bf),
        out_type=jax.ShapeDtypeStruct((NP, nk * FW), odt),
        mesh=plsc.VectorSubcoreMesh(**_MESH),
        scratch_types=[
            pltpu.VMEM((NB, EB), jnp.int32),
            pltpu.VMEM((NB, EB), jnp.int32),
            pltpu.VMEM((NP,), jnp.float32),
            pltpu.VMEM((NP,), jnp.float32),
            pltpu.VMEM((NB, EB), jnp.float32),
            pltpu.VMEM((EB, FW), jnp.float32),
            pltpu.VMEM((EB, FW), jnp.float32),
            pltpu.VMEM((EB, 16), jnp.float32),
            pltpu.VMEM((EB, 16), jnp.float32),
            pltpu.VMEM((EB, FW), odt),
            pltpu.VMEM((FW,), jnp.float32),
            pltpu.SemaphoreType.DMA,
            pltpu.SemaphoreType.DMA,
            pltpu.SemaphoreType.DMA,
            pltpu.SemaphoreType.DMA,
            pltpu.SemaphoreType.DMA,
            pltpu.SemaphoreType.DMA,
            pltpu.VMEM_SHARED((NP, FW), jnp.float32),
            pltpu.VMEM_SHARED((NP, 16), jnp.float32),
        ],
        compiler_params=_SC_PARAMS,
    )()
    return fn(src2d, dst2d, asrc_t, adst_t, tbl, bias2d)


def _sc_decode_body(z_hbm, s2d, d2d, out_hbm, sidx, didx, sr0, dr0, sr1, dr1,
                    pbuf, obuf, gsem0, gsem1):
    cid = lax.axis_index("c")
    sid = lax.axis_index("s")
    g = sid * 2 + cid
    pltpu.sync_copy(s2d.at[g], sidx)
    pltpu.sync_copy(d2d.at[g], didx)
    lanes = lax.iota(jnp.int32, 16)

    def gst(b, sbuf, dbuf, sem):
        pltpu.async_copy(z_hbm.at[sidx.at[b]], sbuf, sem)
        pltpu.async_copy(z_hbm.at[didx.at[b]], dbuf, sem)

    def gwt(b, sbuf, dbuf, sem):
        pltpu.make_async_copy(z_hbm.at[sidx.at[b]], sbuf, sem).wait()
        pltpu.make_async_copy(z_hbm.at[didx.at[b]], dbuf, sem).wait()

    def dot(b, sbuf, dbuf):
        def grp(i, c):
            for l in range(16):
                e = i * 16 + l
                acc = None
                for j in range(256 // 32):
                    sa, sb = plsc.unpack(sbuf[e, pl.ds(j * 32, 32)],
                                         format=_PK)
                    da, db = plsc.unpack(dbuf[e, pl.ds(j * 32, 32)],
                                         format=_PK)
                    t = sa * da + sb * db
                    acc = t if acc is None else acc + t
                pbuf[l, :] = acc
            tot = plsc.load_gather(pbuf, [lanes, jnp.zeros((16,), jnp.int32)])
            for j in range(1, 16):
                tot = tot + plsc.load_gather(
                    pbuf, [lanes, jnp.full((16,), j, jnp.int32)])
            obuf[pl.ds(b * EB + i * 16, 16)] = tot
            return c

        lax.fori_loop(0, EB // 16, grp, 0)

    npair = NBD // 2
    gst(0, sr0, dr0, gsem0)

    def pair(gp, c):
        b0 = 2 * gp
        b1 = b0 + 1
        gst(b1, sr1, dr1, gsem1)
        gwt(b0, sr0, dr0, gsem0)
        dot(b0, sr0, dr0)

        @pl.when(gp < npair - 1)
        def _():
            gst(b0 + 2, sr0, dr0, gsem0)

        gwt(b1, sr1, dr1, gsem1)
        dot(b1, sr1, dr1)
        return c

    lax.fori_loop(0, npair, pair, 0)
    pltpu.sync_copy(obuf, out_hbm.at[pl.ds(g * NBD * EB, NBD * EB)])


def _sc_decode(z, s2d, d2d):
    fn = functools.partial(
        pl.kernel,
        _sc_decode_body,
        out_type=jax.ShapeDtypeStruct((EPD,), jnp.float32),
        mesh=plsc.VectorSubcoreMesh(**_MESH),
        scratch_types=[
            pltpu.VMEM((NBD, EB), jnp.int32),
            pltpu.VMEM((NBD, EB), jnp.int32),
            pltpu.VMEM((EB, 256), jnp.bfloat16),
            pltpu.VMEM((EB, 256), jnp.bfloat16),
            pltpu.VMEM((EB, 256), jnp.bfloat16),
            pltpu.VMEM((EB, 256), jnp.bfloat16),
            pltpu.VMEM((16, 16), jnp.float32),
            pltpu.VMEM((NBD * EB,), jnp.float32),
            pltpu.SemaphoreType.DMA,
            pltpu.SemaphoreType.DMA,
        ],
        compiler_params=_SC_PARAMS,
    )()
    return fn(z, s2d, d2d)


def _block_diag_att(att):
    hh, c = att.shape
    m = jnp.zeros((hh * c, hh), jnp.float32)
    for h in range(hh):
        m = m.at[h * c:(h + 1) * c, h].set(att[h])
    return m


def kernel(x, edge_index, W1, att_src1, att_dst1, b1, W2, att_src2, att_dst2,
           b2):
    ei = edge_index.astype(jnp.int32)
    x_p = jnp.pad(x, ((0, NP - N), (0, 0)))
    loop = jnp.arange(N, dtype=jnp.int32)
    padc = jnp.full((EP - ESL,), N, jnp.int32)
    src2d = jnp.concatenate([ei[0], loop, padc]).reshape(16, NB, EB)
    dst2d = jnp.concatenate([ei[1], loop, padc]).reshape(16, NB, EB)
    tbl1, asrc1, adst1 = _tc_layer(
        x_p, W1, _block_diag_att(att_src1), _block_diag_att(att_dst1),
        nk=8, relu=False)
    asrc1t = asrc1.T.reshape(H * NP)
    adst1t = adst1.T.at[:, N:].set(NEG).reshape(H * NP)
    agg1 = _sc_layer(src2d, dst2d, asrc1t, adst1t, tbl1,
                     b1, nk=8, out_bf=False)

    tbl2, asrc2, adst2 = _tc_layer(
        agg1, W2, _block_diag_att(att_src2),
        _block_diag_att(att_dst2), nk=4, relu=True)
    asrc2t = asrc2.T.reshape(H * NP)
    adst2t = adst2.T.at[:, N:].set(NEG).reshape(H * NP)
    z = _sc_layer(src2d, dst2d, asrc2t, adst2t, tbl2,
                  b2, nk=4, out_bf=True)

    padd = jnp.full((EPD - E0,), N, jnp.int32)
    s2d = jnp.concatenate([ei[0], padd]).reshape(32, NBD, EB)
    d2d = jnp.concatenate([ei[1], padd]).reshape(32, NBD, EB)
    scores = _sc_decode(z, s2d, d2d)
    return scores[:E0]
